# baseline (speedup 1.0000x reference)
import dataclasses
import functools

import jax
import jax.numpy as jnp
from jax import lax
from jax.experimental import pallas as pl
from jax.experimental.pallas import tpu as pltpu
from jax.experimental.pallas import tpu_sc as plsc

_D = 10000
_FIN1 = 4
_CL1_F = 32
_CL1_K = 25
_CL2_F = 64
_CL2_K = 25
_FC1_F = 128
_FC2_F = 10
_B = 16
_NNZ1 = 160000
_NNZ2 = 80000

_NTILES = 16
_LANES = 16


def _cheby_sc_kernel(V, Ch, E, G, K, SUB, EU, RU, NBUF):
    E16 = E // _NTILES
    NCH = E16 // G
    SLAB = V // _NTILES
    NSUB = SLAB // SUB
    assert NCH * G == E16 and NSUB * SUB == SLAB
    assert G % EU == 0 and SUB % RU == 0
    assert NBUF == 1 or NCH % 2 == 0
    WGRP = Ch // _LANES

    mesh = plsc.VectorSubcoreMesh(core_axis_name="c", subcore_axis_name="s")
    cp = pltpu.CompilerParams()
    fields = pltpu.CompilerParams.__dataclass_fields__
    if "needs_layout_passes" in fields:
        cp = dataclasses.replace(cp, needs_layout_passes=False)
    if "use_tc_tiling_on_sc" in fields:
        cp = dataclasses.replace(cp, use_tc_tiling_on_sc=False)

    @functools.partial(
        pl.kernel,
        out_type=jax.ShapeDtypeStruct((K, 2, V, Ch), jnp.float32),
        mesh=mesh,
        compiler_params=cp,
        scratch_types=[
            pltpu.VMEM((NCH, G), jnp.int32),
            pltpu.VMEM((NCH, G), jnp.int32),
            pltpu.VMEM((NCH, G), jnp.float32),
            *[pltpu.VMEM((G, Ch), jnp.float32) for _ in range(NBUF)],
            pltpu.VMEM((SUB, Ch), jnp.float32),
            pltpu.VMEM((SUB, Ch), jnp.float32),
            pltpu.VMEM((SUB, Ch), jnp.float32),
            pltpu.VMEM_SHARED((V, Ch), jnp.float32),
            pltpu.SemaphoreType.DMA,
            pltpu.SemaphoreType.DMA,
        ],
    )
    def kern(x0_hbm, cols_hbm, rows_hbm, vals_hbm, out_hbm,
             cols_v, rows_v, vals_v, *rest):
        gbufs = rest[:NBUF]
        (abuf, b1buf, b0buf, acc_sh, semA, semB) = rest[NBUF:]
        gbufA = gbufs[0]
        gbufB = gbufs[-1]
        c = lax.axis_index("c")
        s = lax.axis_index("s")
        base = s * SLAB

        pltpu.sync_copy(cols_hbm.at[s], cols_v)
        pltpu.sync_copy(rows_hbm.at[s], rows_v)
        pltpu.sync_copy(vals_hbm.at[s], vals_v)

        zeros = jnp.zeros((_LANES,), jnp.float32)

        @pl.loop(0, NSUB)
        def _(i):
            sl = pl.ds(base + i * SUB, SUB)
            pltpu.sync_copy(x0_hbm.at[c, sl], abuf)
            pltpu.sync_copy(abuf, out_hbm.at[0, c, sl])

        plsc.subcore_barrier()

        @pl.loop(1, K)
        def _(kk):
            @pl.loop(0, SUB)
            def _(r):
                for w in range(WGRP):
                    abuf.at[r, pl.ds(w * _LANES, _LANES)][...] = zeros

            @pl.loop(0, NSUB)
            def _(i):
                pltpu.sync_copy(abuf, acc_sh.at[pl.ds(base + i * SUB, SUB)])

            plsc.subcore_barrier()

            src = out_hbm.at[kk - 1, c]

            def issue(j, gb, sem):
                pltpu.async_copy(src.at[cols_v.at[j]], gb, sem)

            def wait(j, gb, sem):
                pltpu.make_async_copy(src.at[cols_v.at[j]], gb, sem).wait()

            def scale_scatter(j, gb):
                @pl.loop(0, G, step=EU)
                def _(e0):
                    for u in range(EU):
                        e = e0 + u
                        vsp = plsc.load_gather(
                            vals_v,
                            [jnp.full((_LANES,), j, jnp.int32),
                             jnp.full((_LANES,), e, jnp.int32)],
                        )
                        for w in range(WGRP):
                            sl = (e, pl.ds(w * _LANES, _LANES))
                            gb.at[*sl][...] = gb.at[*sl][...] * vsp

                pltpu.sync_copy(gb, acc_sh.at[rows_v.at[j]], add=True)

            if NBUF == 2:
                issue(0, gbufA, semA)

                @pl.loop(0, NCH // 2)
                def _(jj):
                    j = 2 * jj
                    wait(j, gbufA, semA)
                    issue(j + 1, gbufB, semB)
                    scale_scatter(j, gbufA)
                    wait(j + 1, gbufB, semB)

                    @pl.when(j + 2 < NCH)
                    def _():
                        issue(j + 2, gbufA, semA)

                    scale_scatter(j + 1, gbufB)
            else:
                @pl.loop(0, NCH)
                def _(j):
                    pltpu.sync_copy(src.at[cols_v.at[j]], gbufA)
                    scale_scatter(j, gbufA)

            plsc.subcore_barrier()

            ca = jnp.where(kk == 1, 1.0, 2.0).astype(jnp.float32)
            cc = jnp.where(kk == 1, 0.0, 1.0).astype(jnp.float32)
            kprev2 = jnp.maximum(kk - 2, 0)

            @pl.loop(0, NSUB)
            def _(i):
                sl = pl.ds(base + i * SUB, SUB)
                pltpu.sync_copy(acc_sh.at[sl], abuf)
                pltpu.sync_copy(out_hbm.at[kk - 1, c, sl], b1buf)
                pltpu.sync_copy(out_hbm.at[kprev2, c, sl], b0buf)

                @pl.loop(0, SUB, step=RU)
                def _(r0):
                    for u in range(RU):
                        r = r0 + u
                        for w in range(WGRP):
                            rsl = (r, pl.ds(w * _LANES, _LANES))
                            abuf.at[*rsl][...] = (
                                ca * abuf.at[*rsl][...]
                                - ca * b1buf.at[*rsl][...]
                                - cc * b0buf.at[*rsl][...]
                            )

                pltpu.sync_copy(abuf, out_hbm.at[kk, c, sl])

            plsc.subcore_barrier()

    return kern


def _cheby_terms(x0, idx, vals_scaled, V, Vp, C, E, G, K, SUB, EU, RU, NBUF):
    Ch = C // 2
    if Vp != V:
        x0 = jnp.pad(x0, ((0, Vp - V), (0, 0)))
    x0s = x0.reshape(Vp, 2, Ch).transpose(1, 0, 2)
    cols3 = idx[1].reshape(_NTILES, -1, G)
    rows3 = idx[0].reshape(_NTILES, -1, G)
    vals3 = vals_scaled.reshape(_NTILES, -1, G)
    kern = _cheby_sc_kernel(Vp, Ch, E, G, K, SUB, EU, RU, NBUF)
    terms = kern(x0s, cols3, rows3, vals3)
    terms = terms.transpose(0, 2, 1, 3).reshape(K, Vp, C)
    return terms[:, :V, :]


def _mm_bias_relu(x, wT, b, blk):
    M, Kd = x.shape
    N = wT.shape[1]
    assert M % blk == 0

    def body(x_ref, w_ref, b_ref, o_ref):
        acc = jnp.dot(x_ref[...], w_ref[...],
                      preferred_element_type=jnp.float32)
        o_ref[...] = jnp.maximum(acc + b_ref[...], 0.0)

    return pl.pallas_call(
        body,
        grid=(M // blk,),
        in_specs=[
            pl.BlockSpec((blk, Kd), lambda i: (i, 0)),
            pl.BlockSpec((Kd, N), lambda i: (0, 0)),
            pl.BlockSpec((1, N), lambda i: (0, 0)),
        ],
        out_specs=pl.BlockSpec((blk, N), lambda i: (i, 0)),
        out_shape=jax.ShapeDtypeStruct((M, N), jnp.float32),
    )(x, wT, b)


def _fc_head(h, w1T, b1, w2T, b2, kblk):
    Bb, Kd = h.shape
    F1 = w1T.shape[1]
    F2 = w2T.shape[1]
    nk = Kd // kblk
    assert nk * kblk == Kd

    def body(h_ref, w1_ref, b1_ref, w2_ref, b2_ref, o_ref, acc):
        i = pl.program_id(0)

        @pl.when(i == 0)
        def _():
            acc[...] = jnp.zeros_like(acc)

        acc[...] += jnp.dot(h_ref[...], w1_ref[...],
                            preferred_element_type=jnp.float32)

        @pl.when(i == nk - 1)
        def _():
            a = jnp.maximum(acc[...] + b1_ref[...], 0.0)
            o_ref[...] = jnp.dot(a, w2_ref[...],
                                 preferred_element_type=jnp.float32) + b2_ref[...]

    return pl.pallas_call(
        body,
        grid=(nk,),
        in_specs=[
            pl.BlockSpec((Bb, kblk), lambda i: (0, i)),
            pl.BlockSpec((kblk, F1), lambda i: (i, 0)),
            pl.BlockSpec((1, F1), lambda i: (0, 0)),
            pl.BlockSpec((F1, F2), lambda i: (0, 0)),
            pl.BlockSpec((1, F2), lambda i: (0, 0)),
        ],
        out_specs=pl.BlockSpec((Bb, F2), lambda i: (0, 0)),
        out_shape=jax.ShapeDtypeStruct((Bb, F2), jnp.float32),
        scratch_shapes=[pltpu.VMEM((Bb, F1), jnp.float32)],
    )(h, w1T, b1, w2T, b2)


def _cheby_layer(x, idx, vals, lmax, W, b, K, Fout, V, Vp, E, G, SUB, EU, RU,
                 NBUF, blk):
    Bb, _, Fin = x.shape
    C = Fin * Bb
    x0 = jnp.transpose(x, (1, 2, 0)).reshape(V, C)
    vals_scaled = vals * (2.0 / lmax)
    terms = _cheby_terms(x0, idx, vals_scaled, V, Vp, C, E, G, K, SUB, EU, RU,
                         NBUF)
    xs = terms.reshape(K, V, Fin, Bb)
    xs = jnp.transpose(xs, (3, 1, 2, 0)).reshape(Bb * V, Fin * K)
    y = _mm_bias_relu(xs, W.T, b.reshape(1, -1), blk)
    return y.reshape(Bb, V, Fout)


def _pool2(x):
    Bb, V, F = x.shape
    x = x.reshape(Bb, V // 2, 2, F)
    return jnp.maximum(x[:, :, 0, :], x[:, :, 1, :])


def kernel(x, L1_indices, L1_values, L2_indices, L2_values, lmax1, lmax2, d,
           cl1_w, cl1_b, cl2_w, cl2_b, fc1_w, fc1_b, fc2_w, fc2_b):
    h = _cheby_layer(x, L1_indices, L1_values, lmax1, cl1_w, cl1_b,
                     _CL1_K, _CL1_F, V=_D, Vp=10240, E=_NNZ1, G=400, SUB=128,
                     EU=5, RU=8, NBUF=1, blk=1000)
    h = _pool2(h)
    h = _cheby_layer(h, L2_indices, L2_values, lmax2, cl2_w, cl2_b,
                     _CL2_K, _CL2_F, V=_D // 2, Vp=5120, E=_NNZ2, G=50,
                     SUB=16, EU=5, RU=2, NBUF=1, blk=1000)
    h = _pool2(h)
    h = h.reshape(_B, -1)
    out = _fc_head(h, fc1_w.T, fc1_b.reshape(1, -1), fc2_w.T,
                   fc2_b.reshape(1, -1), kblk=1280)
    return out


# speedup vs baseline: 1.5118x; 1.5118x over previous
import dataclasses
import functools

import jax
import jax.numpy as jnp
from jax import lax
from jax.experimental import pallas as pl
from jax.experimental.pallas import tpu as pltpu
from jax.experimental.pallas import tpu_sc as plsc

_D = 10000
_FIN1 = 4
_CL1_F = 32
_CL1_K = 25
_CL2_F = 64
_CL2_K = 25
_FC1_F = 128
_FC2_F = 10
_B = 16
_NNZ1 = 160000
_NNZ2 = 80000

_NTILES = 16
_LANES = 16


def _cheby_sc_kernel(V, Ch, E, G, K, SUB, EU, RU, NBUF):
    E16 = E // _NTILES
    NCH = E16 // G
    SLAB = V // _NTILES
    NSUB = SLAB // SUB
    assert NCH * G == E16 and NSUB * SUB == SLAB
    assert G % EU == 0 and SUB % RU == 0
    assert NBUF == 1 or NCH % 2 == 0
    WGRP = Ch // _LANES

    mesh = plsc.VectorSubcoreMesh(core_axis_name="c", subcore_axis_name="s")
    cp = pltpu.CompilerParams()
    fields = pltpu.CompilerParams.__dataclass_fields__
    if "needs_layout_passes" in fields:
        cp = dataclasses.replace(cp, needs_layout_passes=False)
    if "use_tc_tiling_on_sc" in fields:
        cp = dataclasses.replace(cp, use_tc_tiling_on_sc=False)

    @functools.partial(
        pl.kernel,
        out_type=jax.ShapeDtypeStruct((K, 2, V, Ch), jnp.float32),
        mesh=mesh,
        compiler_params=cp,
        scratch_types=[
            pltpu.VMEM((NCH, G), jnp.int32),
            pltpu.VMEM((NCH, G), jnp.int32),
            pltpu.VMEM((NCH, G), jnp.float32),
            *[pltpu.VMEM((G, Ch), jnp.float32) for _ in range(NBUF)],
            pltpu.VMEM((SUB, Ch), jnp.float32),
            pltpu.VMEM((SUB, Ch), jnp.float32),
            pltpu.VMEM((SUB, Ch), jnp.float32),
            pltpu.VMEM_SHARED((V, Ch), jnp.float32),
            pltpu.SemaphoreType.DMA,
            pltpu.SemaphoreType.DMA,
        ],
    )
    def kern(x0_hbm, cols_hbm, rows_hbm, vals_hbm, out_hbm,
             cols_v, rows_v, vals_v, *rest):
        gbufs = rest[:NBUF]
        (abuf, b1buf, b0buf, acc_sh, semA, semB) = rest[NBUF:]
        gbufA = gbufs[0]
        gbufB = gbufs[-1]
        c = lax.axis_index("c")
        s = lax.axis_index("s")
        base = s * SLAB

        pltpu.sync_copy(cols_hbm.at[s], cols_v)
        pltpu.sync_copy(rows_hbm.at[s], rows_v)
        pltpu.sync_copy(vals_hbm.at[s], vals_v)

        zeros = jnp.zeros((_LANES,), jnp.float32)

        @pl.loop(0, NSUB)
        def _(i):
            sl = pl.ds(base + i * SUB, SUB)
            pltpu.sync_copy(x0_hbm.at[c, sl], abuf)
            pltpu.sync_copy(abuf, out_hbm.at[0, c, sl])

        plsc.subcore_barrier()

        @pl.loop(1, K)
        def _(kk):
            @pl.loop(0, SUB)
            def _(r):
                for w in range(WGRP):
                    abuf.at[r, pl.ds(w * _LANES, _LANES)][...] = zeros

            @pl.loop(0, NSUB)
            def _(i):
                pltpu.sync_copy(abuf, acc_sh.at[pl.ds(base + i * SUB, SUB)])

            plsc.subcore_barrier()

            src = out_hbm.at[kk - 1, c]

            def issue(j, gb, sem):
                pltpu.async_copy(src.at[cols_v.at[j]], gb, sem)

            def wait(j, gb, sem):
                pltpu.make_async_copy(src.at[cols_v.at[j]], gb, sem).wait()

            def scale_scatter(j, gb):
                @pl.loop(0, G, step=EU)
                def _(e0):
                    for u in range(EU):
                        e = e0 + u
                        vsp = plsc.load_gather(
                            vals_v,
                            [jnp.full((_LANES,), j, jnp.int32),
                             jnp.full((_LANES,), e, jnp.int32)],
                        )
                        for w in range(WGRP):
                            sl = (e, pl.ds(w * _LANES, _LANES))
                            gb.at[*sl][...] = gb.at[*sl][...] * vsp

                pltpu.sync_copy(gb, acc_sh.at[rows_v.at[j]], add=True)

            if NBUF == 2:
                issue(0, gbufA, semA)

                @pl.loop(0, NCH // 2)
                def _(jj):
                    j = 2 * jj
                    wait(j, gbufA, semA)
                    issue(j + 1, gbufB, semB)
                    scale_scatter(j, gbufA)
                    wait(j + 1, gbufB, semB)

                    @pl.when(j + 2 < NCH)
                    def _():
                        issue(j + 2, gbufA, semA)

                    scale_scatter(j + 1, gbufB)
            else:
                @pl.loop(0, NCH)
                def _(j):
                    pltpu.sync_copy(src.at[cols_v.at[j]], gbufA)
                    scale_scatter(j, gbufA)

            plsc.subcore_barrier()

            ca = jnp.where(kk == 1, 1.0, 2.0).astype(jnp.float32)
            cc = jnp.where(kk == 1, 0.0, 1.0).astype(jnp.float32)
            kprev2 = jnp.maximum(kk - 2, 0)

            @pl.loop(0, NSUB)
            def _(i):
                sl = pl.ds(base + i * SUB, SUB)
                pltpu.sync_copy(acc_sh.at[sl], abuf)
                pltpu.sync_copy(out_hbm.at[kk - 1, c, sl], b1buf)
                pltpu.sync_copy(out_hbm.at[kprev2, c, sl], b0buf)

                @pl.loop(0, SUB, step=RU)
                def _(r0):
                    for u in range(RU):
                        r = r0 + u
                        for w in range(WGRP):
                            rsl = (r, pl.ds(w * _LANES, _LANES))
                            abuf.at[*rsl][...] = (
                                ca * abuf.at[*rsl][...]
                                - ca * b1buf.at[*rsl][...]
                                - cc * b0buf.at[*rsl][...]
                            )

                pltpu.sync_copy(abuf, out_hbm.at[kk, c, sl])

            plsc.subcore_barrier()

    return kern


def _cheby_terms(x0, idx, vals_scaled, V, Vp, C, E, G, K, SUB, EU, RU, NBUF):
    Ch = C // 2
    if Vp != V:
        x0 = jnp.pad(x0, ((0, Vp - V), (0, 0)))
    x0s = x0.reshape(Vp, 2, Ch).transpose(1, 0, 2)
    cols3 = idx[1].reshape(_NTILES, -1, G)
    rows3 = idx[0].reshape(_NTILES, -1, G)
    vals3 = vals_scaled.reshape(_NTILES, -1, G)
    kern = _cheby_sc_kernel(Vp, Ch, E, G, K, SUB, EU, RU, NBUF)
    terms = kern(x0s, cols3, rows3, vals3)
    terms = terms.transpose(0, 2, 1, 3).reshape(K, Vp, C)
    return terms[:, :V, :]


def _mm_bias_relu(x, wT, b, blk):
    M, Kd = x.shape
    N = wT.shape[1]
    assert M % blk == 0

    def body(x_ref, w_ref, b_ref, o_ref):
        acc = jnp.dot(x_ref[...], w_ref[...],
                      preferred_element_type=jnp.float32)
        o_ref[...] = jnp.maximum(acc + b_ref[...], 0.0)

    return pl.pallas_call(
        body,
        grid=(M // blk,),
        in_specs=[
            pl.BlockSpec((blk, Kd), lambda i: (i, 0)),
            pl.BlockSpec((Kd, N), lambda i: (0, 0)),
            pl.BlockSpec((1, N), lambda i: (0, 0)),
        ],
        out_specs=pl.BlockSpec((blk, N), lambda i: (i, 0)),
        out_shape=jax.ShapeDtypeStruct((M, N), jnp.float32),
    )(x, wT, b)


def _fc_head(h, w1T, b1, w2T, b2, kblk):
    Bb, Kd = h.shape
    F1 = w1T.shape[1]
    F2 = w2T.shape[1]
    nk = Kd // kblk
    assert nk * kblk == Kd

    def body(h_ref, w1_ref, b1_ref, w2_ref, b2_ref, o_ref, acc):
        i = pl.program_id(0)

        @pl.when(i == 0)
        def _():
            acc[...] = jnp.zeros_like(acc)

        acc[...] += jnp.dot(h_ref[...], w1_ref[...],
                            preferred_element_type=jnp.float32)

        @pl.when(i == nk - 1)
        def _():
            a = jnp.maximum(acc[...] + b1_ref[...], 0.0)
            o_ref[...] = jnp.dot(a, w2_ref[...],
                                 preferred_element_type=jnp.float32) + b2_ref[...]

    return pl.pallas_call(
        body,
        grid=(nk,),
        in_specs=[
            pl.BlockSpec((Bb, kblk), lambda i: (0, i)),
            pl.BlockSpec((kblk, F1), lambda i: (i, 0)),
            pl.BlockSpec((1, F1), lambda i: (0, 0)),
            pl.BlockSpec((F1, F2), lambda i: (0, 0)),
            pl.BlockSpec((1, F2), lambda i: (0, 0)),
        ],
        out_specs=pl.BlockSpec((Bb, F2), lambda i: (0, 0)),
        out_shape=jax.ShapeDtypeStruct((Bb, F2), jnp.float32),
        scratch_shapes=[pltpu.VMEM((Bb, F1), jnp.float32)],
    )(h, w1T, b1, w2T, b2)


def _cheby_layer(x, idx, vals, lmax, W, b, K, Fout, V, Vp, E, G, SUB, EU, RU,
                 NBUF, blk):
    Bb, _, Fin = x.shape
    C = Fin * Bb
    x0 = jnp.transpose(x, (1, 2, 0)).reshape(V, C)
    vals_scaled = vals * (2.0 / lmax)
    terms = _cheby_terms(x0, idx, vals_scaled, V, Vp, C, E, G, K, SUB, EU, RU,
                         NBUF)
    xs = terms.reshape(K, V, Fin, Bb)
    xs = jnp.transpose(xs, (3, 1, 2, 0)).reshape(Bb * V, Fin * K)
    y = _mm_bias_relu(xs, W.T, b.reshape(1, -1), blk)
    return y.reshape(Bb, V, Fout)


def _pool2(x):
    Bb, V, F = x.shape
    x = x.reshape(Bb, V // 2, 2, F)
    return jnp.maximum(x[:, :, 0, :], x[:, :, 1, :])


def kernel(x, L1_indices, L1_values, L2_indices, L2_values, lmax1, lmax2, d,
           cl1_w, cl1_b, cl2_w, cl2_b, fc1_w, fc1_b, fc2_w, fc2_b):
    h = _cheby_layer(x, L1_indices, L1_values, lmax1, cl1_w, cl1_b,
                     _CL1_K, _CL1_F, V=_D, Vp=10240, E=_NNZ1, G=400, SUB=128,
                     EU=1, RU=1, NBUF=1, blk=1000)
    h = _pool2(h)
    h = _cheby_layer(h, L2_indices, L2_values, lmax2, cl2_w, cl2_b,
                     _CL2_K, _CL2_F, V=_D // 2, Vp=5120, E=_NNZ2, G=50,
                     SUB=16, EU=1, RU=1, NBUF=1, blk=1000)
    h = _pool2(h)
    h = h.reshape(_B, -1)
    out = _fc_head(h, fc1_w.T, fc1_b.reshape(1, -1), fc2_w.T,
                   fc2_b.reshape(1, -1), kblk=1280)
    return out
